# Initial kernel scaffold; baseline (speedup 1.0000x reference)
import jax
import jax.numpy as jnp
from jax.experimental import pallas as pl


def kernel(entity_emb, user_emb, latent_emb, edge_index, edge_type, ui_indices, ui_values, weight, disen_weight_att, kg_W_r):
    raise NotImplementedError("write your pallas kernel here")


# speedup vs baseline: 2.4009x; 2.4009x over previous
import functools
import math

import jax
import jax.numpy as jnp
from jax import lax
from jax.experimental import pallas as pl
from jax.experimental.pallas import tpu as pltpu
from jax.experimental.pallas import tpu_sc as plsc

N = 10000
U = 20000
C = 128
E = 320000
NNZ = 1000000
F = 4
NW = 32
EW = E // NW
G = 80
NNZW = 31280
NNZP = NNZW * NW

_i16 = lambda: lax.iota(jnp.int32, 16)
_f16 = lambda v: jnp.full((16,), v, jnp.float32)
_c16 = lambda v: jnp.full((16,), v, jnp.int32)


def _sc_mesh():
    return plsc.VectorSubcoreMesh(core_axis_name="c", subcore_axis_name="s")


def _wid():
    return lax.axis_index("s") * 2 + lax.axis_index("c")


def _zero_shared_2d(sh_ref, zbuf, rows_total, s):
    nj = zbuf.shape[1] // 16

    @pl.when(s == 0)
    def _():
        @pl.loop(0, G)
        def _(r):
            for j in range(nj):
                zbuf[r, pl.ds(j * 16, 16)] = _f16(0.0)

        @pl.loop(0, rows_total // G)
        def _(i):
            pltpu.sync_copy(zbuf, sh_ref.at[pl.ds(i * G, G)])


def _zero_shared_1d(sh_ref, zbuf1, rows_total, s):
    @pl.when(s == 0)
    def _():
        @pl.loop(0, G // 16)
        def _(j):
            zbuf1[pl.ds(j * 16, 16)] = _f16(0.0)

        @pl.loop(0, rows_total // G)
        def _(i):
            pltpu.sync_copy(zbuf1, sh_ref.at[pl.ds(i * G, G)])


def _combine(p):
    return pl.pallas_call(
        lambda p_ref, o_ref: o_ref.__setitem__(
            (Ellipsis,), p_ref[0] + p_ref[1]),
        out_shape=jax.ShapeDtypeStruct((10, 1000), jnp.float32),
    )(p.reshape(2, 10, 1000)).reshape(N)


def _k1_body(e_ref, w_ref, kwr_ref, t16_ref, nrm_ref):
    q = jnp.dot(e_ref[...], kwr_ref[...], preferred_element_type=jnp.float32)
    wk = jnp.dot(w_ref[...], kwr_ref[...], preferred_element_type=jnp.float32)
    s = lax.dot_general(q, wk, (((1,), (1,)), ((), ())),
                        preferred_element_type=jnp.float32)
    t16_ref[...] = jnp.exp(s * (1.0 / 16.0))

    @pl.when(pl.program_id(0) == 0)
    def _():
        n = jnp.sum(w_ref[...] * w_ref[...], axis=1, keepdims=True)
        row = lax.dot_general(
            jnp.ones((8, 1), jnp.float32)[:1], n,
            (((1,), (1,)), ((), ())), preferred_element_type=jnp.float32)
        nrm_ref[...] = jnp.concatenate([row, jnp.zeros((7, 16), jnp.float32)], 0)


def _k1(entity_emb, w16, kg_W_r):
    return pl.pallas_call(
        _k1_body,
        grid=(10,),
        in_specs=[
            pl.BlockSpec((1000, C), lambda i: (i, 0)),
            pl.BlockSpec((16, C), lambda i: (0, 0)),
            pl.BlockSpec((C, C), lambda i: (0, 0)),
        ],
        out_specs=[
            pl.BlockSpec((1000, 16), lambda i: (i, 0)),
            pl.BlockSpec((8, 16), lambda i: (0, 0)),
        ],
        out_shape=[
            jax.ShapeDtypeStruct((N, 16), jnp.float32),
            jax.ShapeDtypeStruct((8, 16), jnp.float32),
        ],
    )(entity_emb, w16, kg_W_r)


def _k2_body(head_hbm, tail_hbm, typ_hbm, ent_hbm, t16f_hbm,
             hd_hbm, exr_hbm, segr_hbm,
             hidx, tidx, typv, sidx, hrows, trows, exrv, hdv, zbuf1,
             segr_sh, sem1, sem2, sem3):
    c = lax.axis_index("c")
    s = lax.axis_index("s")
    wid = s * 2 + c
    _zero_shared_1d(segr_sh, zbuf1, N, s)
    plsc.subcore_barrier()
    base0 = wid * EW

    @pl.loop(0, EW // G)
    def _(i):
        b = base0 + i * G
        pltpu.sync_copy(head_hbm.at[pl.ds(b, G)], hidx)
        pltpu.sync_copy(tail_hbm.at[pl.ds(b, G)], tidx)
        pltpu.sync_copy(typ_hbm.at[pl.ds(b, G)], typv)
        cp1 = pltpu.async_copy(ent_hbm.at[hidx], hrows, sem1)
        cp2 = pltpu.async_copy(ent_hbm.at[tidx], trows, sem2)

        @pl.loop(0, G // 16)
        def _(g):
            d = pl.ds(g * 16, 16)
            sidx[d] = hidx[d] * 16 + typv[d] - 1

        pltpu.async_copy(t16f_hbm.at[sidx], exrv, sem3).wait()
        cp1.wait()
        cp2.wait()

        @pl.loop(0, G // 16)
        def _(g):
            d = pl.ds(g * 16, 16)
            lanes = _i16()
            hd16 = _f16(0.0)
            for l in range(16):
                e = g * 16 + l
                acc = _f16(0.0)
                for j in range(C // 16):
                    dj = pl.ds(j * 16, 16)
                    acc = acc + hrows[e, dj] * trows[e, dj]
                parts = [acc[k] for k in range(16)]
                while len(parts) > 1:
                    parts = [parts[k] + parts[k + 1]
                             for k in range(0, len(parts), 2)]
                hd16 = jnp.where(lanes == l, parts[0], hd16)
            hdv[d] = hd16

        pltpu.sync_copy(exrv, segr_sh.at[hidx], add=True)
        pltpu.sync_copy(hdv, hd_hbm.at[pl.ds(b, G)])
        pltpu.sync_copy(exrv, exr_hbm.at[pl.ds(b, G)])

    plsc.subcore_barrier()

    @pl.when(s == 0)
    def _():
        pltpu.sync_copy(segr_sh, segr_hbm.at[c])


def _k2(head, tail, typ, ent, t16):
    f = pl.kernel(
        _k2_body,
        out_type=(
            jax.ShapeDtypeStruct((E,), jnp.float32),
            jax.ShapeDtypeStruct((E,), jnp.float32),
            jax.ShapeDtypeStruct((2, N), jnp.float32),
        ),
        mesh=_sc_mesh(),
        scratch_types=(
            pltpu.VMEM((G,), jnp.int32),
            pltpu.VMEM((G,), jnp.int32),
            pltpu.VMEM((G,), jnp.int32),
            pltpu.VMEM((G,), jnp.int32),
            pltpu.VMEM((G, C), jnp.float32),
            pltpu.VMEM((G, C), jnp.float32),
            pltpu.VMEM((G,), jnp.float32),
            pltpu.VMEM((G,), jnp.float32),
            pltpu.VMEM((G,), jnp.float32),
            pltpu.VMEM_SHARED((N,), jnp.float32),
            pltpu.SemaphoreType.DMA,
            pltpu.SemaphoreType.DMA,
            pltpu.SemaphoreType.DMA,
        ),
    )
    return f(head, tail, typ, ent, t16)


def _k4_body(head_hbm, typ_hbm, hd_hbm, exr_hbm, segr_hbm, nrmf_hbm,
             ext_hbm, segt_hbm,
             hidx, nidx, hdv, exv, extv, srv, nrv, zbuf1,
             segt_sh, sem1, sem2):
    c = lax.axis_index("c")
    s = lax.axis_index("s")
    wid = s * 2 + c
    _zero_shared_1d(segt_sh, zbuf1, N, s)
    plsc.subcore_barrier()
    base0 = wid * EW

    @pl.loop(0, EW // G)
    def _(i):
        b = base0 + i * G
        pltpu.sync_copy(head_hbm.at[pl.ds(b, G)], hidx)
        pltpu.sync_copy(typ_hbm.at[pl.ds(b, G)], nidx)
        pltpu.sync_copy(hd_hbm.at[pl.ds(b, G)], hdv)
        pltpu.sync_copy(exr_hbm.at[pl.ds(b, G)], exv)

        @pl.loop(0, G // 16)
        def _(g):
            d = pl.ds(g * 16, 16)
            nidx[d] = nidx[d] - 1

        cp1 = pltpu.async_copy(segr_hbm.at[hidx], srv, sem1)
        cp2 = pltpu.async_copy(nrmf_hbm.at[nidx], nrv, sem2)
        cp1.wait()
        cp2.wait()

        @pl.loop(0, G // 16)
        def _(g):
            d = pl.ds(g * 16, 16)
            a = exv[d] / srv[d]
            extv[d] = jnp.exp(hdv[d] + a * a * nrv[d])

        pltpu.sync_copy(extv, segt_sh.at[hidx], add=True)
        pltpu.sync_copy(extv, ext_hbm.at[pl.ds(b, G)])

    plsc.subcore_barrier()

    @pl.when(s == 0)
    def _():
        pltpu.sync_copy(segt_sh, segt_hbm.at[c])


def _k4(head, typ, hd, exr, segr, nrmf):
    f = pl.kernel(
        _k4_body,
        out_type=(
            jax.ShapeDtypeStruct((E,), jnp.float32),
            jax.ShapeDtypeStruct((2, N), jnp.float32),
        ),
        mesh=_sc_mesh(),
        scratch_types=(
            pltpu.VMEM((G,), jnp.int32),
            pltpu.VMEM((G,), jnp.int32),
            pltpu.VMEM((G,), jnp.float32),
            pltpu.VMEM((G,), jnp.float32),
            pltpu.VMEM((G,), jnp.float32),
            pltpu.VMEM((G,), jnp.float32),
            pltpu.VMEM((G,), jnp.float32),
            pltpu.VMEM((G,), jnp.float32),
            pltpu.VMEM_SHARED((N,), jnp.float32),
            pltpu.SemaphoreType.DMA,
            pltpu.SemaphoreType.DMA,
        ),
    )
    return f(head, typ, hd, exr, segr, nrmf)


def _k6a_body(head_hbm, tail_hbm, typ_hbm, ext_hbm, segt_hbm, wflat_hbm,
              ent_hbm, eagg_hbm,
              hidx, tidx, typv, extv, srv, rows, wloc, zbuf,
              eagg_sh, sem1, sem2):
    c = lax.axis_index("c")
    s = lax.axis_index("s")
    wid = s * 2 + c
    pltpu.sync_copy(wflat_hbm, wloc)
    _zero_shared_2d(eagg_sh, zbuf, N, s)
    plsc.subcore_barrier()
    base0 = wid * EW

    @pl.loop(0, EW // G)
    def _(i):
        b = base0 + i * G
        pltpu.sync_copy(head_hbm.at[pl.ds(b, G)], hidx)
        pltpu.sync_copy(tail_hbm.at[pl.ds(b, G)], tidx)
        pltpu.sync_copy(typ_hbm.at[pl.ds(b, G)], typv)
        pltpu.sync_copy(ext_hbm.at[pl.ds(b, G)], extv)
        cp1 = pltpu.async_copy(ent_hbm.at[tidx], rows, sem1)
        cp2 = pltpu.async_copy(segt_hbm.at[hidx], srv, sem2)
        cp1.wait()
        cp2.wait()

        @pl.loop(0, G // 16)
        def _(g):
            d = pl.ds(g * 16, 16)
            kg16 = extv[d] / srv[d]
            wb16 = (typv[d] - 1) * C
            for l in range(16):
                e = g * 16 + l
                kg_e = kg16[l]
                wb_e = wb16[l]
                for j in range(C // 16):
                    dj = pl.ds(j * 16, 16)
                    w = wloc[pl.ds(wb_e + j * 16, 16)]
                    rows[e, dj] = rows[e, dj] * w * kg_e

        pltpu.sync_copy(rows, eagg_sh.at[hidx], add=True)

    plsc.subcore_barrier()

    @pl.when(s == 0)
    def _():
        pltpu.sync_copy(eagg_sh, eagg_hbm.at[c])


def _k6a(head, tail, typ, ext, segt, wflat, ent):
    f = pl.kernel(
        _k6a_body,
        out_type=jax.ShapeDtypeStruct((2, N, C), jnp.float32),
        mesh=_sc_mesh(),
        scratch_types=(
            pltpu.VMEM((G,), jnp.int32),
            pltpu.VMEM((G,), jnp.int32),
            pltpu.VMEM((G,), jnp.int32),
            pltpu.VMEM((G,), jnp.float32),
            pltpu.VMEM((G,), jnp.float32),
            pltpu.VMEM((G, C), jnp.float32),
            pltpu.VMEM((16 * C,), jnp.float32),
            pltpu.VMEM((G, C), jnp.float32),
            pltpu.VMEM_SHARED((N, C), jnp.float32),
            pltpu.SemaphoreType.DMA,
            pltpu.SemaphoreType.DMA,
        ),
    )
    return f(head, tail, typ, ext, segt, wflat, ent)


UH = U // 2
UHP = 10080


def _k6b_body(off, rr_hbm, cc_hbm, val_hbm, ent_hbm, uagg_hbm,
              ridxb, ridx2, cidx, vv, rows, zbuf, uagg_sh, sem1):
    c = lax.axis_index("c")
    s = lax.axis_index("s")
    wid = s * 2 + c
    _zero_shared_2d(uagg_sh, zbuf, UHP, s)
    plsc.subcore_barrier()
    base0 = wid * NNZW

    @pl.loop(0, NNZW // G)
    def _(i):
        b = base0 + i * G
        pltpu.sync_copy(rr_hbm.at[pl.ds(b, G)], ridxb)
        pltpu.sync_copy(cc_hbm.at[pl.ds(b, G)], cidx)
        pltpu.sync_copy(val_hbm.at[pl.ds(b, G)], vv)
        cp1 = pltpu.async_copy(ent_hbm.at[cidx], rows, sem1)

        @pl.loop(0, G // 16)
        def _(g):
            d = pl.ds(g * 16, 16)
            r = ridxb[d] - off
            ok = (r >= 0) & (r < UH)
            ridx2[d] = jnp.where(ok, r, UH)

        cp1.wait()

        @pl.loop(0, G // 16)
        def _(g):
            vvec = vv[pl.ds(g * 16, 16)]
            for l in range(16):
                e = g * 16 + l
                v_e = vvec[l]
                for j in range(C // 16):
                    dj = pl.ds(j * 16, 16)
                    rows[e, dj] = rows[e, dj] * v_e

        pltpu.sync_copy(rows, uagg_sh.at[ridx2], add=True)

    plsc.subcore_barrier()

    @pl.when(s == 0)
    def _():
        pltpu.sync_copy(uagg_sh.at[pl.ds(0, UH)], uagg_hbm.at[c])


def _k6b(rr, cc, vals, ent, off):
    f = pl.kernel(
        functools.partial(_k6b_body, off),
        out_type=jax.ShapeDtypeStruct((2, UH, C), jnp.float32),
        mesh=_sc_mesh(),
        scratch_types=(
            pltpu.VMEM((G,), jnp.int32),
            pltpu.VMEM((G,), jnp.int32),
            pltpu.VMEM((G,), jnp.int32),
            pltpu.VMEM((G,), jnp.float32),
            pltpu.VMEM((G, C), jnp.float32),
            pltpu.VMEM((G, C), jnp.float32),
            pltpu.VMEM_SHARED((UHP, C), jnp.float32),
            pltpu.SemaphoreType.DMA,
        ),
    )
    return f(rr, cc, vals, ent)


def _k7e_body(eagg_ref, eres_ref, ent_ref, eout_ref):
    agg = eagg_ref[0] + eagg_ref[1]
    nrm = jnp.maximum(
        jnp.sqrt(jnp.sum(agg * agg, axis=1, keepdims=True)), 1e-12)
    ent = agg / nrm
    ent_ref[...] = ent
    eout_ref[...] = eres_ref[...] + ent


def _k7e(eagg_p, eres):
    return pl.pallas_call(
        _k7e_body,
        grid=(10,),
        in_specs=[
            pl.BlockSpec((2, 1000, C), lambda i: (0, i, 0)),
            pl.BlockSpec((1000, C), lambda i: (i, 0)),
        ],
        out_specs=[
            pl.BlockSpec((1000, C), lambda i: (i, 0)),
            pl.BlockSpec((1000, C), lambda i: (i, 0)),
        ],
        out_shape=[
            jax.ShapeDtypeStruct((N, C), jnp.float32),
            jax.ShapeDtypeStruct((N, C), jnp.float32),
        ],
    )(eagg_p, eres)


def _masked_softmax(x, valid):
    m = lax.broadcasted_iota(jnp.int32, x.shape, 1) < valid
    xm = jnp.where(m, x, -1e30)
    ex = jnp.exp(xm - jnp.max(xm, axis=1, keepdims=True))
    ex = jnp.where(m, ex, 0.0)
    return ex / jnp.sum(ex, axis=1, keepdims=True)


def _k7u_body(up_ref, usr_ref, lat_ref, att_ref, w_ref, ures_ref,
              uout_ref, uresout_ref):
    uagg = up_ref[0] + up_ref[1]
    usr = usr_ref[...]
    logits = lax.dot_general(usr, lat_ref[...], (((1,), (1,)), ((), ())),
                             preferred_element_type=jnp.float32)
    score = _masked_softmax(logits, F)
    disen = jnp.dot(_masked_softmax(att_ref[...], 9), w_ref[...],
                    preferred_element_type=jnp.float32)
    mix = jnp.dot(score, disen, preferred_element_type=jnp.float32)
    ua = uagg * mix + uagg
    nrm = jnp.maximum(jnp.sqrt(jnp.sum(ua * ua, axis=1, keepdims=True)), 1e-12)
    un = ua / nrm
    uout_ref[...] = un
    uresout_ref[...] = ures_ref[...] + un


def _k7u(up, usr, lat8, att16, w16, ures):
    B = 2000
    return pl.pallas_call(
        _k7u_body,
        grid=(UH // B,),
        in_specs=[
            pl.BlockSpec((2, B, C), lambda i: (0, i, 0)),
            pl.BlockSpec((B, C), lambda i: (i, 0)),
            pl.BlockSpec((8, C), lambda i: (0, 0)),
            pl.BlockSpec((8, 16), lambda i: (0, 0)),
            pl.BlockSpec((16, C), lambda i: (0, 0)),
            pl.BlockSpec((B, C), lambda i: (i, 0)),
        ],
        out_specs=[
            pl.BlockSpec((B, C), lambda i: (i, 0)),
            pl.BlockSpec((B, C), lambda i: (i, 0)),
        ],
        out_shape=[
            jax.ShapeDtypeStruct((UH, C), jnp.float32),
            jax.ShapeDtypeStruct((UH, C), jnp.float32),
        ],
    )(up, usr, lat8, att16, w16, ures)


def kernel(entity_emb, user_emb, latent_emb, edge_index, edge_type,
           ui_indices, ui_values, weight, disen_weight_att, kg_W_r):
    head = edge_index[0]
    tail = edge_index[1]
    typ = edge_type
    w16 = jnp.concatenate([weight, jnp.zeros((7, C), jnp.float32)], 0)
    att16 = jnp.zeros((8, 16), jnp.float32).at[:F, :9].set(disen_weight_att)
    lat8 = jnp.concatenate([latent_emb, jnp.zeros((F, C), jnp.float32)], 0)
    wflat = w16.reshape(16 * C)

    pad = NNZP - NNZ
    rr = jnp.concatenate([ui_indices[0], jnp.zeros((pad,), jnp.int32)])
    cc = jnp.concatenate([ui_indices[1], jnp.zeros((pad,), jnp.int32)])
    vals = jnp.concatenate([ui_values, jnp.zeros((pad,), jnp.float32)])

    t16, nrm16 = _k1(entity_emb, w16, kg_W_r)
    hd, exr, segr_p = _k2(head, tail, typ, entity_emb, t16.reshape(-1))
    segr = _combine(segr_p)
    ext, segt_p = _k4(head, typ, hd, exr, segr, nrm16[0])
    segt = _combine(segt_p)

    ent = entity_emb
    usr = user_emb
    eres = entity_emb
    ures = user_emb
    for _ in range(2):
        eagg_p = _k6a(head, tail, typ, ext, segt, wflat, ent)
        ulo_p = _k6b(rr, cc, vals, ent, 0)
        uhi_p = _k6b(rr, cc, vals, ent, UH)
        ent, eres = _k7e(eagg_p, eres)
        usr_lo, ures_lo = _k7u(ulo_p, usr[:UH], lat8, att16, w16, ures[:UH])
        usr_hi, ures_hi = _k7u(uhi_p, usr[UH:], lat8, att16, w16, ures[UH:])
        usr = jnp.concatenate([usr_lo, usr_hi], 0)
        ures = jnp.concatenate([ures_lo, ures_hi], 0)
    return (eres, ures)


# speedup vs baseline: 2.8095x; 1.1702x over previous
import functools
import math

import jax
import jax.numpy as jnp
from jax import lax
from jax.experimental import pallas as pl
from jax.experimental.pallas import tpu as pltpu
from jax.experimental.pallas import tpu_sc as plsc

N = 10000
U = 20000
C = 128
E = 320000
NNZ = 1000000
F = 4
NW = 32
EW = E // NW
G = 80
NNZW = 31360
NNZP = NNZW * NW
G2 = 80

_i16 = lambda: lax.iota(jnp.int32, 16)
_f16 = lambda v: jnp.full((16,), v, jnp.float32)
_c16 = lambda v: jnp.full((16,), v, jnp.int32)


def _sc_mesh():
    return plsc.VectorSubcoreMesh(core_axis_name="c", subcore_axis_name="s")


def _wid():
    return lax.axis_index("s") * 2 + lax.axis_index("c")


def _zero_shared_2d(sh_ref, zbuf, rows_total, s):
    nj = zbuf.shape[1] // 16
    gz = zbuf.shape[0]

    @pl.when(s == 0)
    def _():
        @pl.loop(0, gz)
        def _(r):
            for j in range(nj):
                zbuf[r, pl.ds(j * 16, 16)] = _f16(0.0)

        @pl.loop(0, rows_total // gz)
        def _(i):
            pltpu.sync_copy(zbuf, sh_ref.at[pl.ds(i * gz, gz)])


def _zero_shared_1d(sh_ref, zbuf1, rows_total, s):
    @pl.when(s == 0)
    def _():
        @pl.loop(0, G // 16)
        def _(j):
            zbuf1[pl.ds(j * 16, 16)] = _f16(0.0)

        @pl.loop(0, rows_total // G)
        def _(i):
            pltpu.sync_copy(zbuf1, sh_ref.at[pl.ds(i * G, G)])


def _combine(p):
    return pl.pallas_call(
        lambda p_ref, o_ref: o_ref.__setitem__(
            (Ellipsis,), p_ref[0] + p_ref[1]),
        out_shape=jax.ShapeDtypeStruct((10, 1000), jnp.float32),
    )(p.reshape(2, 10, 1000)).reshape(N)


def _k1_body(e_ref, w_ref, kwr_ref, t16_ref, nrm_ref):
    q = jnp.dot(e_ref[...], kwr_ref[...], preferred_element_type=jnp.float32)
    wk = jnp.dot(w_ref[...], kwr_ref[...], preferred_element_type=jnp.float32)
    s = lax.dot_general(q, wk, (((1,), (1,)), ((), ())),
                        preferred_element_type=jnp.float32)
    t16_ref[...] = jnp.exp(s * (1.0 / 16.0))

    @pl.when(pl.program_id(0) == 0)
    def _():
        n = jnp.sum(w_ref[...] * w_ref[...], axis=1, keepdims=True)
        row = lax.dot_general(
            jnp.ones((8, 1), jnp.float32)[:1], n,
            (((1,), (1,)), ((), ())), preferred_element_type=jnp.float32)
        nrm_ref[...] = jnp.concatenate([row, jnp.zeros((7, 16), jnp.float32)], 0)


def _k1(entity_emb, w16, kg_W_r):
    return pl.pallas_call(
        _k1_body,
        grid=(10,),
        in_specs=[
            pl.BlockSpec((1000, C), lambda i: (i, 0)),
            pl.BlockSpec((16, C), lambda i: (0, 0)),
            pl.BlockSpec((C, C), lambda i: (0, 0)),
        ],
        out_specs=[
            pl.BlockSpec((1000, 16), lambda i: (i, 0)),
            pl.BlockSpec((8, 16), lambda i: (0, 0)),
        ],
        out_shape=[
            jax.ShapeDtypeStruct((N, 16), jnp.float32),
            jax.ShapeDtypeStruct((8, 16), jnp.float32),
        ],
    )(entity_emb, w16, kg_W_r)


def _k2_body(head_hbm, tail_hbm, typ_hbm, ent_hbm, t16f_hbm,
             hd_hbm, exr_hbm, segr_hbm,
             hidx, tidx, typv, sidx, hrows, trows, exrv, hdv, zbuf1,
             segr_sh, sem1, sem2, sem3):
    c = lax.axis_index("c")
    s = lax.axis_index("s")
    wid = s * 2 + c
    _zero_shared_1d(segr_sh, zbuf1, N, s)
    plsc.subcore_barrier()
    base0 = wid * EW

    @pl.loop(0, EW // G)
    def _(i):
        b = base0 + i * G
        pltpu.sync_copy(head_hbm.at[pl.ds(b, G)], hidx)
        pltpu.sync_copy(tail_hbm.at[pl.ds(b, G)], tidx)
        pltpu.sync_copy(typ_hbm.at[pl.ds(b, G)], typv)
        cp1 = pltpu.async_copy(ent_hbm.at[hidx], hrows, sem1)
        cp2 = pltpu.async_copy(ent_hbm.at[tidx], trows, sem2)

        @pl.loop(0, G // 16)
        def _(g):
            d = pl.ds(g * 16, 16)
            sidx[d] = hidx[d] * 16 + typv[d] - 1

        pltpu.async_copy(t16f_hbm.at[sidx], exrv, sem3).wait()
        cp1.wait()
        cp2.wait()

        @pl.loop(0, G // 16)
        def _(g):
            d = pl.ds(g * 16, 16)
            lanes = _i16()
            hd16 = _f16(0.0)
            for l in range(16):
                e = g * 16 + l
                acc = _f16(0.0)
                for j in range(C // 16):
                    dj = pl.ds(j * 16, 16)
                    acc = acc + hrows[e, dj] * trows[e, dj]
                parts = [acc[k] for k in range(16)]
                while len(parts) > 1:
                    parts = [parts[k] + parts[k + 1]
                             for k in range(0, len(parts), 2)]
                hd16 = jnp.where(lanes == l, parts[0], hd16)
            hdv[d] = hd16

        pltpu.sync_copy(exrv, segr_sh.at[hidx], add=True)
        pltpu.sync_copy(hdv, hd_hbm.at[pl.ds(b, G)])
        pltpu.sync_copy(exrv, exr_hbm.at[pl.ds(b, G)])

    plsc.subcore_barrier()

    @pl.when(s == 0)
    def _():
        pltpu.sync_copy(segr_sh, segr_hbm.at[c])


def _k2(head, tail, typ, ent, t16):
    f = pl.kernel(
        _k2_body,
        out_type=(
            jax.ShapeDtypeStruct((E,), jnp.float32),
            jax.ShapeDtypeStruct((E,), jnp.float32),
            jax.ShapeDtypeStruct((2, N), jnp.float32),
        ),
        mesh=_sc_mesh(),
        scratch_types=(
            pltpu.VMEM((G,), jnp.int32),
            pltpu.VMEM((G,), jnp.int32),
            pltpu.VMEM((G,), jnp.int32),
            pltpu.VMEM((G,), jnp.int32),
            pltpu.VMEM((G, C), jnp.float32),
            pltpu.VMEM((G, C), jnp.float32),
            pltpu.VMEM((G,), jnp.float32),
            pltpu.VMEM((G,), jnp.float32),
            pltpu.VMEM((G,), jnp.float32),
            pltpu.VMEM_SHARED((N,), jnp.float32),
            pltpu.SemaphoreType.DMA,
            pltpu.SemaphoreType.DMA,
            pltpu.SemaphoreType.DMA,
        ),
    )
    return f(head, tail, typ, ent, t16)


def _k4_body(head_hbm, typ_hbm, hd_hbm, exr_hbm, segr_hbm, nrmf_hbm,
             ext_hbm, segt_hbm,
             hidx, nidx, hdv, exv, extv, srv, nrv, zbuf1,
             segt_sh, sem1, sem2):
    c = lax.axis_index("c")
    s = lax.axis_index("s")
    wid = s * 2 + c
    _zero_shared_1d(segt_sh, zbuf1, N, s)
    plsc.subcore_barrier()
    base0 = wid * EW

    @pl.loop(0, EW // G)
    def _(i):
        b = base0 + i * G
        pltpu.sync_copy(head_hbm.at[pl.ds(b, G)], hidx)
        pltpu.sync_copy(typ_hbm.at[pl.ds(b, G)], nidx)
        pltpu.sync_copy(hd_hbm.at[pl.ds(b, G)], hdv)
        pltpu.sync_copy(exr_hbm.at[pl.ds(b, G)], exv)

        @pl.loop(0, G // 16)
        def _(g):
            d = pl.ds(g * 16, 16)
            nidx[d] = nidx[d] - 1

        cp1 = pltpu.async_copy(segr_hbm.at[hidx], srv, sem1)
        cp2 = pltpu.async_copy(nrmf_hbm.at[nidx], nrv, sem2)
        cp1.wait()
        cp2.wait()

        @pl.loop(0, G // 16)
        def _(g):
            d = pl.ds(g * 16, 16)
            a = exv[d] / srv[d]
            extv[d] = jnp.exp(hdv[d] + a * a * nrv[d])

        pltpu.sync_copy(extv, segt_sh.at[hidx], add=True)
        pltpu.sync_copy(extv, ext_hbm.at[pl.ds(b, G)])

    plsc.subcore_barrier()

    @pl.when(s == 0)
    def _():
        pltpu.sync_copy(segt_sh, segt_hbm.at[c])


def _k4(head, typ, hd, exr, segr, nrmf):
    f = pl.kernel(
        _k4_body,
        out_type=(
            jax.ShapeDtypeStruct((E,), jnp.float32),
            jax.ShapeDtypeStruct((2, N), jnp.float32),
        ),
        mesh=_sc_mesh(),
        scratch_types=(
            pltpu.VMEM((G,), jnp.int32),
            pltpu.VMEM((G,), jnp.int32),
            pltpu.VMEM((G,), jnp.float32),
            pltpu.VMEM((G,), jnp.float32),
            pltpu.VMEM((G,), jnp.float32),
            pltpu.VMEM((G,), jnp.float32),
            pltpu.VMEM((G,), jnp.float32),
            pltpu.VMEM((G,), jnp.float32),
            pltpu.VMEM_SHARED((N,), jnp.float32),
            pltpu.SemaphoreType.DMA,
            pltpu.SemaphoreType.DMA,
        ),
    )
    return f(head, typ, hd, exr, segr, nrmf)


def _k6a_body(head_hbm, tail_hbm, typ_hbm, ext_hbm, segt_hbm, wflat_hbm,
              ent_hbm, eagg_hbm,
              hidx, tidx, typv, extv, srv, rows, wloc, zbuf,
              eagg_sh, sem1, sem2):
    c = lax.axis_index("c")
    s = lax.axis_index("s")
    wid = s * 2 + c
    pltpu.sync_copy(wflat_hbm, wloc)
    _zero_shared_2d(eagg_sh, zbuf, N, s)
    plsc.subcore_barrier()
    base0 = wid * EW

    @pl.loop(0, EW // G)
    def _(i):
        b = base0 + i * G
        pltpu.sync_copy(head_hbm.at[pl.ds(b, G)], hidx)
        pltpu.sync_copy(tail_hbm.at[pl.ds(b, G)], tidx)
        pltpu.sync_copy(typ_hbm.at[pl.ds(b, G)], typv)
        pltpu.sync_copy(ext_hbm.at[pl.ds(b, G)], extv)
        cp1 = pltpu.async_copy(ent_hbm.at[tidx], rows, sem1)
        cp2 = pltpu.async_copy(segt_hbm.at[hidx], srv, sem2)
        cp1.wait()
        cp2.wait()

        @pl.loop(0, G // 16)
        def _(g):
            d = pl.ds(g * 16, 16)
            kg16 = extv[d] / srv[d]
            wb16 = (typv[d] - 1) * C
            for l in range(16):
                e = g * 16 + l
                kg_e = kg16[l]
                wb_e = wb16[l]
                for j in range(C // 16):
                    dj = pl.ds(j * 16, 16)
                    w = wloc[pl.ds(wb_e + j * 16, 16)]
                    rows[e, dj] = rows[e, dj] * w * kg_e

        pltpu.sync_copy(rows, eagg_sh.at[hidx], add=True)

    plsc.subcore_barrier()

    @pl.when(s == 0)
    def _():
        pltpu.sync_copy(eagg_sh, eagg_hbm.at[c])


def _k6a(head, tail, typ, ext, segt, wflat, ent):
    f = pl.kernel(
        _k6a_body,
        out_type=jax.ShapeDtypeStruct((2, N, C), jnp.float32),
        mesh=_sc_mesh(),
        scratch_types=(
            pltpu.VMEM((G,), jnp.int32),
            pltpu.VMEM((G,), jnp.int32),
            pltpu.VMEM((G,), jnp.int32),
            pltpu.VMEM((G,), jnp.float32),
            pltpu.VMEM((G,), jnp.float32),
            pltpu.VMEM((G, C), jnp.float32),
            pltpu.VMEM((16 * C,), jnp.float32),
            pltpu.VMEM((G, C), jnp.float32),
            pltpu.VMEM_SHARED((N, C), jnp.float32),
            pltpu.SemaphoreType.DMA,
            pltpu.SemaphoreType.DMA,
        ),
    )
    return f(head, tail, typ, ext, segt, wflat, ent)


UH = U // 2
UHP = 10080


def _k6b_body(off, rr_hbm, cc_hbm, val_hbm, ent_hbm, uagg_hbm,
              ridxb0, ridxb1, ridx2, cidx0, cidx1, vv0, vv1,
              rows0, rows1, zbuf, uagg_sh, sem0, sem1):
    c = lax.axis_index("c")
    s = lax.axis_index("s")
    wid = s * 2 + c
    _zero_shared_2d(uagg_sh, zbuf, UHP, s)
    plsc.subcore_barrier()
    base0 = wid * NNZW
    nch = NNZW // G2
    ridxb = (ridxb0, ridxb1)
    cidx = (cidx0, cidx1)
    vv = (vv0, vv1)
    rows = (rows0, rows1)
    sems = (sem0, sem1)

    def fetch(i, b):
        bb = base0 + i * G2
        pltpu.sync_copy(rr_hbm.at[pl.ds(bb, G2)], ridxb[b])
        pltpu.sync_copy(cc_hbm.at[pl.ds(bb, G2)], cidx[b])
        pltpu.sync_copy(val_hbm.at[pl.ds(bb, G2)], vv[b])
        pltpu.async_copy(ent_hbm.at[cidx[b]], rows[b], sems[b])

    fetch(0, 0)

    @pl.loop(0, nch // 2)
    def _(k):
        for b in range(2):
            i = k * 2 + b

            @pl.when(i + 1 < nch)
            def _():
                fetch(i + 1, 1 - b)

            pltpu.make_async_copy(
                ent_hbm.at[cidx[b]], rows[b], sems[b]).wait()

            @pl.loop(0, G2 // 16)
            def _(g):
                d = pl.ds(g * 16, 16)
                r = ridxb[b][d] - off
                ok = (r >= 0) & (r < UH)
                ridx2[d] = jnp.where(ok, r, UH)

            @pl.loop(0, G2 // 16)
            def _(g):
                vvec = vv[b][pl.ds(g * 16, 16)]
                for l in range(16):
                    e = g * 16 + l
                    v_e = vvec[l]
                    for j in range(C // 16):
                        dj = pl.ds(j * 16, 16)
                        rows[b][e, dj] = rows[b][e, dj] * v_e

            pltpu.sync_copy(rows[b], uagg_sh.at[ridx2], add=True)

    plsc.subcore_barrier()

    @pl.when(s == 0)
    def _():
        pltpu.sync_copy(uagg_sh.at[pl.ds(0, UH)], uagg_hbm.at[c])


def _k6b(rr, cc, vals, ent, off):
    f = pl.kernel(
        functools.partial(_k6b_body, off),
        out_type=jax.ShapeDtypeStruct((2, UH, C), jnp.float32),
        mesh=_sc_mesh(),
        scratch_types=(
            pltpu.VMEM((G2,), jnp.int32),
            pltpu.VMEM((G2,), jnp.int32),
            pltpu.VMEM((G2,), jnp.int32),
            pltpu.VMEM((G2,), jnp.int32),
            pltpu.VMEM((G2,), jnp.int32),
            pltpu.VMEM((G2,), jnp.float32),
            pltpu.VMEM((G2,), jnp.float32),
            pltpu.VMEM((G2, C), jnp.float32),
            pltpu.VMEM((G2, C), jnp.float32),
            pltpu.VMEM((G2, C), jnp.float32),
            pltpu.VMEM_SHARED((UHP, C), jnp.float32),
            pltpu.SemaphoreType.DMA,
            pltpu.SemaphoreType.DMA,
        ),
    )
    return f(rr, cc, vals, ent)


def _k7e_body(eagg_ref, eres_ref, ent_ref, eout_ref):
    agg = eagg_ref[0] + eagg_ref[1]
    nrm = jnp.maximum(
        jnp.sqrt(jnp.sum(agg * agg, axis=1, keepdims=True)), 1e-12)
    ent = agg / nrm
    ent_ref[...] = ent
    eout_ref[...] = eres_ref[...] + ent


def _k7e(eagg_p, eres):
    return pl.pallas_call(
        _k7e_body,
        grid=(10,),
        in_specs=[
            pl.BlockSpec((2, 1000, C), lambda i: (0, i, 0)),
            pl.BlockSpec((1000, C), lambda i: (i, 0)),
        ],
        out_specs=[
            pl.BlockSpec((1000, C), lambda i: (i, 0)),
            pl.BlockSpec((1000, C), lambda i: (i, 0)),
        ],
        out_shape=[
            jax.ShapeDtypeStruct((N, C), jnp.float32),
            jax.ShapeDtypeStruct((N, C), jnp.float32),
        ],
    )(eagg_p, eres)


def _masked_softmax(x, valid):
    m = lax.broadcasted_iota(jnp.int32, x.shape, 1) < valid
    xm = jnp.where(m, x, -1e30)
    ex = jnp.exp(xm - jnp.max(xm, axis=1, keepdims=True))
    ex = jnp.where(m, ex, 0.0)
    return ex / jnp.sum(ex, axis=1, keepdims=True)


def _k7u_body(up_ref, usr_ref, lat_ref, att_ref, w_ref, ures_ref,
              uout_ref, uresout_ref):
    uagg = up_ref[0] + up_ref[1]
    usr = usr_ref[...]
    logits = lax.dot_general(usr, lat_ref[...], (((1,), (1,)), ((), ())),
                             preferred_element_type=jnp.float32)
    score = _masked_softmax(logits, F)
    disen = jnp.dot(_masked_softmax(att_ref[...], 9), w_ref[...],
                    preferred_element_type=jnp.float32)
    mix = jnp.dot(score, disen, preferred_element_type=jnp.float32)
    ua = uagg * mix + uagg
    nrm = jnp.maximum(jnp.sqrt(jnp.sum(ua * ua, axis=1, keepdims=True)), 1e-12)
    un = ua / nrm
    uout_ref[...] = un
    uresout_ref[...] = ures_ref[...] + un


def _k7u(up, usr, lat8, att16, w16, ures):
    B = 2000
    return pl.pallas_call(
        _k7u_body,
        grid=(UH // B,),
        in_specs=[
            pl.BlockSpec((2, B, C), lambda i: (0, i, 0)),
            pl.BlockSpec((B, C), lambda i: (i, 0)),
            pl.BlockSpec((8, C), lambda i: (0, 0)),
            pl.BlockSpec((8, 16), lambda i: (0, 0)),
            pl.BlockSpec((16, C), lambda i: (0, 0)),
            pl.BlockSpec((B, C), lambda i: (i, 0)),
        ],
        out_specs=[
            pl.BlockSpec((B, C), lambda i: (i, 0)),
            pl.BlockSpec((B, C), lambda i: (i, 0)),
        ],
        out_shape=[
            jax.ShapeDtypeStruct((UH, C), jnp.float32),
            jax.ShapeDtypeStruct((UH, C), jnp.float32),
        ],
    )(up, usr, lat8, att16, w16, ures)


def kernel(entity_emb, user_emb, latent_emb, edge_index, edge_type,
           ui_indices, ui_values, weight, disen_weight_att, kg_W_r):
    head = edge_index[0]
    tail = edge_index[1]
    typ = edge_type
    w16 = jnp.concatenate([weight, jnp.zeros((7, C), jnp.float32)], 0)
    att16 = jnp.zeros((8, 16), jnp.float32).at[:F, :9].set(disen_weight_att)
    lat8 = jnp.concatenate([latent_emb, jnp.zeros((F, C), jnp.float32)], 0)
    wflat = w16.reshape(16 * C)

    pad = NNZP - NNZ
    rr = jnp.concatenate([ui_indices[0], jnp.zeros((pad,), jnp.int32)])
    cc = jnp.concatenate([ui_indices[1], jnp.zeros((pad,), jnp.int32)])
    vals = jnp.concatenate([ui_values, jnp.zeros((pad,), jnp.float32)])
    assert NNZW % G2 == 0 and (NNZW // G2) % 2 == 0 and UHP % G2 == 0

    t16, nrm16 = _k1(entity_emb, w16, kg_W_r)
    hd, exr, segr_p = _k2(head, tail, typ, entity_emb, t16.reshape(-1))
    segr = _combine(segr_p)
    ext, segt_p = _k4(head, typ, hd, exr, segr, nrm16[0])
    segt = _combine(segt_p)

    ent = entity_emb
    usr = user_emb
    eres = entity_emb
    ures = user_emb
    for _ in range(2):
        eagg_p = _k6a(head, tail, typ, ext, segt, wflat, ent)
        ulo_p = _k6b(rr, cc, vals, ent, 0)
        uhi_p = _k6b(rr, cc, vals, ent, UH)
        ent, eres = _k7e(eagg_p, eres)
        usr_lo, ures_lo = _k7u(ulo_p, usr[:UH], lat8, att16, w16, ures[:UH])
        usr_hi, ures_hi = _k7u(uhi_p, usr[UH:], lat8, att16, w16, ures[UH:])
        usr = jnp.concatenate([usr_lo, usr_hi], 0)
        ures = jnp.concatenate([ures_lo, ures_hi], 0)
    return (eres, ures)


# speedup vs baseline: 2.8862x; 1.0273x over previous
import functools
import math

import jax
import jax.numpy as jnp
from jax import lax
from jax.experimental import pallas as pl
from jax.experimental.pallas import tpu as pltpu
from jax.experimental.pallas import tpu_sc as plsc

N = 10000
U = 20000
C = 128
E = 320000
NNZ = 1000000
F = 4
NW = 32
EW = E // NW
G = 80
NNZW = 31360
NNZP = NNZW * NW
G2 = 80

_i16 = lambda: lax.iota(jnp.int32, 16)
_f16 = lambda v: jnp.full((16,), v, jnp.float32)
_c16 = lambda v: jnp.full((16,), v, jnp.int32)


def _sc_mesh():
    return plsc.VectorSubcoreMesh(core_axis_name="c", subcore_axis_name="s")


def _wid():
    return lax.axis_index("s") * 2 + lax.axis_index("c")


def _zero_shared_2d(sh_ref, zbuf, rows_total, s):
    nj = zbuf.shape[1] // 16
    gz = zbuf.shape[0]

    @pl.when(s == 0)
    def _():
        @pl.loop(0, gz)
        def _(r):
            for j in range(nj):
                zbuf[r, pl.ds(j * 16, 16)] = _f16(0.0)

        @pl.loop(0, rows_total // gz)
        def _(i):
            pltpu.sync_copy(zbuf, sh_ref.at[pl.ds(i * gz, gz)])


def _zero_shared_1d(sh_ref, zbuf1, rows_total, s):
    @pl.when(s == 0)
    def _():
        @pl.loop(0, G // 16)
        def _(j):
            zbuf1[pl.ds(j * 16, 16)] = _f16(0.0)

        @pl.loop(0, rows_total // G)
        def _(i):
            pltpu.sync_copy(zbuf1, sh_ref.at[pl.ds(i * G, G)])


def _combine(p):
    return pl.pallas_call(
        lambda p_ref, o_ref: o_ref.__setitem__(
            (Ellipsis,), p_ref[0] + p_ref[1]),
        out_shape=jax.ShapeDtypeStruct((10, 1000), jnp.float32),
    )(p.reshape(2, 10, 1000)).reshape(N)


def _k1_body(e_ref, w_ref, kwr_ref, t16_ref, nrm_ref):
    q = jnp.dot(e_ref[...], kwr_ref[...], preferred_element_type=jnp.float32)
    wk = jnp.dot(w_ref[...], kwr_ref[...], preferred_element_type=jnp.float32)
    s = lax.dot_general(q, wk, (((1,), (1,)), ((), ())),
                        preferred_element_type=jnp.float32)
    t16_ref[...] = jnp.exp(s * (1.0 / 16.0))

    @pl.when(pl.program_id(0) == 0)
    def _():
        n = jnp.sum(w_ref[...] * w_ref[...], axis=1, keepdims=True)
        row = lax.dot_general(
            jnp.ones((8, 1), jnp.float32)[:1], n,
            (((1,), (1,)), ((), ())), preferred_element_type=jnp.float32)
        nrm_ref[...] = jnp.concatenate([row, jnp.zeros((7, 16), jnp.float32)], 0)


def _k1(entity_emb, w16, kg_W_r):
    return pl.pallas_call(
        _k1_body,
        grid=(10,),
        in_specs=[
            pl.BlockSpec((1000, C), lambda i: (i, 0)),
            pl.BlockSpec((16, C), lambda i: (0, 0)),
            pl.BlockSpec((C, C), lambda i: (0, 0)),
        ],
        out_specs=[
            pl.BlockSpec((1000, 16), lambda i: (i, 0)),
            pl.BlockSpec((8, 16), lambda i: (0, 0)),
        ],
        out_shape=[
            jax.ShapeDtypeStruct((N, 16), jnp.float32),
            jax.ShapeDtypeStruct((8, 16), jnp.float32),
        ],
    )(entity_emb, w16, kg_W_r)


def _k2_body(head_hbm, tail_hbm, typ_hbm, ent_hbm, t16f_hbm,
             hd_hbm, exr_hbm, segr_hbm,
             hidx, tidx, typv, sidx, hrows, trows, exrv, hdv, zbuf1,
             segr_sh, sem1, sem2, sem3):
    c = lax.axis_index("c")
    s = lax.axis_index("s")
    wid = s * 2 + c
    _zero_shared_1d(segr_sh, zbuf1, N, s)
    plsc.subcore_barrier()
    base0 = wid * EW

    @pl.loop(0, EW // G)
    def _(i):
        b = base0 + i * G
        pltpu.sync_copy(head_hbm.at[pl.ds(b, G)], hidx)
        pltpu.sync_copy(tail_hbm.at[pl.ds(b, G)], tidx)
        pltpu.sync_copy(typ_hbm.at[pl.ds(b, G)], typv)
        cp1 = pltpu.async_copy(ent_hbm.at[hidx], hrows, sem1)
        cp2 = pltpu.async_copy(ent_hbm.at[tidx], trows, sem2)

        @pl.loop(0, G // 16)
        def _(g):
            d = pl.ds(g * 16, 16)
            sidx[d] = hidx[d] * 16 + typv[d] - 1

        pltpu.async_copy(t16f_hbm.at[sidx], exrv, sem3).wait()
        cp1.wait()
        cp2.wait()

        @pl.loop(0, G // 16)
        def _(g):
            d = pl.ds(g * 16, 16)
            lanes = _i16()
            hd16 = _f16(0.0)
            for l in range(16):
                e = g * 16 + l
                acc = _f16(0.0)
                for j in range(C // 16):
                    dj = pl.ds(j * 16, 16)
                    acc = acc + hrows[e, dj] * trows[e, dj]
                parts = [acc[k] for k in range(16)]
                while len(parts) > 1:
                    parts = [parts[k] + parts[k + 1]
                             for k in range(0, len(parts), 2)]
                hd16 = jnp.where(lanes == l, parts[0], hd16)
            hdv[d] = hd16

        pltpu.sync_copy(exrv, segr_sh.at[hidx], add=True)
        pltpu.sync_copy(hdv, hd_hbm.at[pl.ds(b, G)])
        pltpu.sync_copy(exrv, exr_hbm.at[pl.ds(b, G)])

    plsc.subcore_barrier()

    @pl.when(s == 0)
    def _():
        pltpu.sync_copy(segr_sh, segr_hbm.at[c])


def _k2(head, tail, typ, ent, t16):
    f = pl.kernel(
        _k2_body,
        out_type=(
            jax.ShapeDtypeStruct((E,), jnp.float32),
            jax.ShapeDtypeStruct((E,), jnp.float32),
            jax.ShapeDtypeStruct((2, N), jnp.float32),
        ),
        mesh=_sc_mesh(),
        scratch_types=(
            pltpu.VMEM((G,), jnp.int32),
            pltpu.VMEM((G,), jnp.int32),
            pltpu.VMEM((G,), jnp.int32),
            pltpu.VMEM((G,), jnp.int32),
            pltpu.VMEM((G, C), jnp.float32),
            pltpu.VMEM((G, C), jnp.float32),
            pltpu.VMEM((G,), jnp.float32),
            pltpu.VMEM((G,), jnp.float32),
            pltpu.VMEM((G,), jnp.float32),
            pltpu.VMEM_SHARED((N,), jnp.float32),
            pltpu.SemaphoreType.DMA,
            pltpu.SemaphoreType.DMA,
            pltpu.SemaphoreType.DMA,
        ),
    )
    return f(head, tail, typ, ent, t16)


def _k4_body(head_hbm, typ_hbm, hd_hbm, exr_hbm, segr_hbm, nrmf_hbm,
             ext_hbm, segt_hbm,
             hidx, nidx, hdv, exv, extv, srv, nrv, zbuf1,
             segt_sh, sem1, sem2):
    c = lax.axis_index("c")
    s = lax.axis_index("s")
    wid = s * 2 + c
    _zero_shared_1d(segt_sh, zbuf1, N, s)
    plsc.subcore_barrier()
    base0 = wid * EW

    @pl.loop(0, EW // G)
    def _(i):
        b = base0 + i * G
        pltpu.sync_copy(head_hbm.at[pl.ds(b, G)], hidx)
        pltpu.sync_copy(typ_hbm.at[pl.ds(b, G)], nidx)
        pltpu.sync_copy(hd_hbm.at[pl.ds(b, G)], hdv)
        pltpu.sync_copy(exr_hbm.at[pl.ds(b, G)], exv)

        @pl.loop(0, G // 16)
        def _(g):
            d = pl.ds(g * 16, 16)
            nidx[d] = nidx[d] - 1

        cp1 = pltpu.async_copy(segr_hbm.at[hidx], srv, sem1)
        cp2 = pltpu.async_copy(nrmf_hbm.at[nidx], nrv, sem2)
        cp1.wait()
        cp2.wait()

        @pl.loop(0, G // 16)
        def _(g):
            d = pl.ds(g * 16, 16)
            a = exv[d] / srv[d]
            extv[d] = jnp.exp(hdv[d] + a * a * nrv[d])

        pltpu.sync_copy(extv, segt_sh.at[hidx], add=True)
        pltpu.sync_copy(extv, ext_hbm.at[pl.ds(b, G)])

    plsc.subcore_barrier()

    @pl.when(s == 0)
    def _():
        pltpu.sync_copy(segt_sh, segt_hbm.at[c])


def _k4(head, typ, hd, exr, segr, nrmf):
    f = pl.kernel(
        _k4_body,
        out_type=(
            jax.ShapeDtypeStruct((E,), jnp.float32),
            jax.ShapeDtypeStruct((2, N), jnp.float32),
        ),
        mesh=_sc_mesh(),
        scratch_types=(
            pltpu.VMEM((G,), jnp.int32),
            pltpu.VMEM((G,), jnp.int32),
            pltpu.VMEM((G,), jnp.float32),
            pltpu.VMEM((G,), jnp.float32),
            pltpu.VMEM((G,), jnp.float32),
            pltpu.VMEM((G,), jnp.float32),
            pltpu.VMEM((G,), jnp.float32),
            pltpu.VMEM((G,), jnp.float32),
            pltpu.VMEM_SHARED((N,), jnp.float32),
            pltpu.SemaphoreType.DMA,
            pltpu.SemaphoreType.DMA,
        ),
    )
    return f(head, typ, hd, exr, segr, nrmf)


def _k6a_body(head_hbm, tail_hbm, typ_hbm, ext_hbm, segt_hbm, wflat_hbm,
              ent_hbm, eagg_hbm,
              hidx0, hidx1, tidx0, tidx1, typv0, typv1, extv0, extv1,
              srv0, srv1, rows0, rows1, wloc, zbuf,
              eagg_sh, sem0, sem1, sem2, sem3):
    c = lax.axis_index("c")
    s = lax.axis_index("s")
    wid = s * 2 + c
    pltpu.sync_copy(wflat_hbm, wloc)
    _zero_shared_2d(eagg_sh, zbuf, N, s)
    plsc.subcore_barrier()
    base0 = wid * EW
    nch = EW // G
    hidx = (hidx0, hidx1)
    tidx = (tidx0, tidx1)
    typv = (typv0, typv1)
    extv = (extv0, extv1)
    srv = (srv0, srv1)
    rows = (rows0, rows1)
    rsem = (sem0, sem1)
    ssem = (sem2, sem3)

    def fetch(i, b):
        bb = base0 + i * G
        pltpu.sync_copy(head_hbm.at[pl.ds(bb, G)], hidx[b])
        pltpu.sync_copy(tail_hbm.at[pl.ds(bb, G)], tidx[b])
        pltpu.sync_copy(typ_hbm.at[pl.ds(bb, G)], typv[b])
        pltpu.sync_copy(ext_hbm.at[pl.ds(bb, G)], extv[b])
        pltpu.async_copy(ent_hbm.at[tidx[b]], rows[b], rsem[b])
        pltpu.async_copy(segt_hbm.at[hidx[b]], srv[b], ssem[b])

    def work(b):
        pltpu.make_async_copy(ent_hbm.at[tidx[b]], rows[b], rsem[b]).wait()
        pltpu.make_async_copy(segt_hbm.at[hidx[b]], srv[b], ssem[b]).wait()

        @pl.loop(0, G // 16)
        def _(g):
            d = pl.ds(g * 16, 16)
            kg16 = extv[b][d] / srv[b][d]
            wb16 = (typv[b][d] - 1) * C
            for l in range(16):
                e = g * 16 + l
                kg_e = kg16[l]
                wb_e = wb16[l]
                for j in range(C // 16):
                    dj = pl.ds(j * 16, 16)
                    w = wloc[pl.ds(wb_e + j * 16, 16)]
                    rows[b][e, dj] = rows[b][e, dj] * w * kg_e

        pltpu.sync_copy(rows[b], eagg_sh.at[hidx[b]], add=True)

    fetch(0, 0)

    @pl.loop(0, nch // 2)
    def _(k):
        for b in range(2):
            i = k * 2 + b

            @pl.when(i + 1 < nch)
            def _():
                fetch(i + 1, 1 - b)

            work(b)

    if nch % 2 == 1:
        work(0)

    plsc.subcore_barrier()

    @pl.when(s == 0)
    def _():
        pltpu.sync_copy(eagg_sh, eagg_hbm.at[c])


def _k6a(head, tail, typ, ext, segt, wflat, ent):
    f = pl.kernel(
        _k6a_body,
        out_type=jax.ShapeDtypeStruct((2, N, C), jnp.float32),
        mesh=_sc_mesh(),
        scratch_types=(
            pltpu.VMEM((G,), jnp.int32),
            pltpu.VMEM((G,), jnp.int32),
            pltpu.VMEM((G,), jnp.int32),
            pltpu.VMEM((G,), jnp.int32),
            pltpu.VMEM((G,), jnp.int32),
            pltpu.VMEM((G,), jnp.int32),
            pltpu.VMEM((G,), jnp.float32),
            pltpu.VMEM((G,), jnp.float32),
            pltpu.VMEM((G,), jnp.float32),
            pltpu.VMEM((G,), jnp.float32),
            pltpu.VMEM((G, C), jnp.float32),
            pltpu.VMEM((G, C), jnp.float32),
            pltpu.VMEM((16 * C,), jnp.float32),
            pltpu.VMEM((G, C), jnp.float32),
            pltpu.VMEM_SHARED((N, C), jnp.float32),
            pltpu.SemaphoreType.DMA,
            pltpu.SemaphoreType.DMA,
            pltpu.SemaphoreType.DMA,
            pltpu.SemaphoreType.DMA,
        ),
    )
    return f(head, tail, typ, ext, segt, wflat, ent)


UH = U // 2
UHP = 10080


def _k6b_body(off, rr_hbm, cc_hbm, val_hbm, ent_hbm, uagg_hbm,
              ridxb0, ridxb1, ridx2, cidx0, cidx1, vv0, vv1,
              rows0, rows1, zbuf, uagg_sh, sem0, sem1):
    c = lax.axis_index("c")
    s = lax.axis_index("s")
    wid = s * 2 + c
    _zero_shared_2d(uagg_sh, zbuf, UHP, s)
    plsc.subcore_barrier()
    base0 = wid * NNZW
    nch = NNZW // G2
    ridxb = (ridxb0, ridxb1)
    cidx = (cidx0, cidx1)
    vv = (vv0, vv1)
    rows = (rows0, rows1)
    sems = (sem0, sem1)

    def fetch(i, b):
        bb = base0 + i * G2
        pltpu.sync_copy(rr_hbm.at[pl.ds(bb, G2)], ridxb[b])
        pltpu.sync_copy(cc_hbm.at[pl.ds(bb, G2)], cidx[b])
        pltpu.sync_copy(val_hbm.at[pl.ds(bb, G2)], vv[b])
        pltpu.async_copy(ent_hbm.at[cidx[b]], rows[b], sems[b])

    fetch(0, 0)

    @pl.loop(0, nch // 2)
    def _(k):
        for b in range(2):
            i = k * 2 + b

            @pl.when(i + 1 < nch)
            def _():
                fetch(i + 1, 1 - b)

            pltpu.make_async_copy(
                ent_hbm.at[cidx[b]], rows[b], sems[b]).wait()

            @pl.loop(0, G2 // 16)
            def _(g):
                d = pl.ds(g * 16, 16)
                r = ridxb[b][d] - off
                ok = (r >= 0) & (r < UH)
                ridx2[d] = jnp.where(ok, r, UH)

            @pl.loop(0, G2 // 16)
            def _(g):
                vvec = vv[b][pl.ds(g * 16, 16)]
                for l in range(16):
                    e = g * 16 + l
                    v_e = vvec[l]
                    for j in range(C // 16):
                        dj = pl.ds(j * 16, 16)
                        rows[b][e, dj] = rows[b][e, dj] * v_e

            pltpu.sync_copy(rows[b], uagg_sh.at[ridx2], add=True)

    plsc.subcore_barrier()

    @pl.when(s == 0)
    def _():
        pltpu.sync_copy(uagg_sh.at[pl.ds(0, UH)], uagg_hbm.at[c])


def _k6b(rr, cc, vals, ent, off):
    f = pl.kernel(
        functools.partial(_k6b_body, off),
        out_type=jax.ShapeDtypeStruct((2, UH, C), jnp.float32),
        mesh=_sc_mesh(),
        scratch_types=(
            pltpu.VMEM((G2,), jnp.int32),
            pltpu.VMEM((G2,), jnp.int32),
            pltpu.VMEM((G2,), jnp.int32),
            pltpu.VMEM((G2,), jnp.int32),
            pltpu.VMEM((G2,), jnp.int32),
            pltpu.VMEM((G2,), jnp.float32),
            pltpu.VMEM((G2,), jnp.float32),
            pltpu.VMEM((G2, C), jnp.float32),
            pltpu.VMEM((G2, C), jnp.float32),
            pltpu.VMEM((G2, C), jnp.float32),
            pltpu.VMEM_SHARED((UHP, C), jnp.float32),
            pltpu.SemaphoreType.DMA,
            pltpu.SemaphoreType.DMA,
        ),
    )
    return f(rr, cc, vals, ent)


def _k7e_body(eagg_ref, eres_ref, ent_ref, eout_ref):
    agg = eagg_ref[0] + eagg_ref[1]
    nrm = jnp.maximum(
        jnp.sqrt(jnp.sum(agg * agg, axis=1, keepdims=True)), 1e-12)
    ent = agg / nrm
    ent_ref[...] = ent
    eout_ref[...] = eres_ref[...] + ent


def _k7e(eagg_p, eres):
    return pl.pallas_call(
        _k7e_body,
        grid=(10,),
        in_specs=[
            pl.BlockSpec((2, 1000, C), lambda i: (0, i, 0)),
            pl.BlockSpec((1000, C), lambda i: (i, 0)),
        ],
        out_specs=[
            pl.BlockSpec((1000, C), lambda i: (i, 0)),
            pl.BlockSpec((1000, C), lambda i: (i, 0)),
        ],
        out_shape=[
            jax.ShapeDtypeStruct((N, C), jnp.float32),
            jax.ShapeDtypeStruct((N, C), jnp.float32),
        ],
    )(eagg_p, eres)


def _masked_softmax(x, valid):
    m = lax.broadcasted_iota(jnp.int32, x.shape, 1) < valid
    xm = jnp.where(m, x, -1e30)
    ex = jnp.exp(xm - jnp.max(xm, axis=1, keepdims=True))
    ex = jnp.where(m, ex, 0.0)
    return ex / jnp.sum(ex, axis=1, keepdims=True)


def _k7u_body(up_ref, usr_ref, lat_ref, att_ref, w_ref, ures_ref,
              uout_ref, uresout_ref):
    uagg = up_ref[0] + up_ref[1]
    usr = usr_ref[...]
    logits = lax.dot_general(usr, lat_ref[...], (((1,), (1,)), ((), ())),
                             preferred_element_type=jnp.float32)
    score = _masked_softmax(logits, F)
    disen = jnp.dot(_masked_softmax(att_ref[...], 9), w_ref[...],
                    preferred_element_type=jnp.float32)
    mix = jnp.dot(score, disen, preferred_element_type=jnp.float32)
    ua = uagg * mix + uagg
    nrm = jnp.maximum(jnp.sqrt(jnp.sum(ua * ua, axis=1, keepdims=True)), 1e-12)
    un = ua / nrm
    uout_ref[...] = un
    uresout_ref[...] = ures_ref[...] + un


def _k7u(up, usr, lat8, att16, w16, ures):
    B = 2000
    return pl.pallas_call(
        _k7u_body,
        grid=(UH // B,),
        in_specs=[
            pl.BlockSpec((2, B, C), lambda i: (0, i, 0)),
            pl.BlockSpec((B, C), lambda i: (i, 0)),
            pl.BlockSpec((8, C), lambda i: (0, 0)),
            pl.BlockSpec((8, 16), lambda i: (0, 0)),
            pl.BlockSpec((16, C), lambda i: (0, 0)),
            pl.BlockSpec((B, C), lambda i: (i, 0)),
        ],
        out_specs=[
            pl.BlockSpec((B, C), lambda i: (i, 0)),
            pl.BlockSpec((B, C), lambda i: (i, 0)),
        ],
        out_shape=[
            jax.ShapeDtypeStruct((UH, C), jnp.float32),
            jax.ShapeDtypeStruct((UH, C), jnp.float32),
        ],
    )(up, usr, lat8, att16, w16, ures)


def kernel(entity_emb, user_emb, latent_emb, edge_index, edge_type,
           ui_indices, ui_values, weight, disen_weight_att, kg_W_r):
    head = edge_index[0]
    tail = edge_index[1]
    typ = edge_type
    w16 = jnp.concatenate([weight, jnp.zeros((7, C), jnp.float32)], 0)
    att16 = jnp.zeros((8, 16), jnp.float32).at[:F, :9].set(disen_weight_att)
    lat8 = jnp.concatenate([latent_emb, jnp.zeros((F, C), jnp.float32)], 0)
    wflat = w16.reshape(16 * C)

    pad = NNZP - NNZ
    rr = jnp.concatenate([ui_indices[0], jnp.zeros((pad,), jnp.int32)])
    cc = jnp.concatenate([ui_indices[1], jnp.zeros((pad,), jnp.int32)])
    vals = jnp.concatenate([ui_values, jnp.zeros((pad,), jnp.float32)])
    assert NNZW % G2 == 0 and (NNZW // G2) % 2 == 0 and UHP % G2 == 0

    t16, nrm16 = _k1(entity_emb, w16, kg_W_r)
    hd, exr, segr_p = _k2(head, tail, typ, entity_emb, t16.reshape(-1))
    segr = _combine(segr_p)
    ext, segt_p = _k4(head, typ, hd, exr, segr, nrm16[0])
    segt = _combine(segt_p)

    ent = entity_emb
    usr = user_emb
    eres = entity_emb
    ures = user_emb
    for _ in range(2):
        eagg_p = _k6a(head, tail, typ, ext, segt, wflat, ent)
        ulo_p = _k6b(rr, cc, vals, ent, 0)
        uhi_p = _k6b(rr, cc, vals, ent, UH)
        ent, eres = _k7e(eagg_p, eres)
        usr_lo, ures_lo = _k7u(ulo_p, usr[:UH], lat8, att16, w16, ures[:UH])
        usr_hi, ures_hi = _k7u(uhi_p, usr[UH:], lat8, att16, w16, ures[UH:])
        usr = jnp.concatenate([usr_lo, usr_hi], 0)
        ures = jnp.concatenate([ures_lo, ures_hi], 0)
    return (eres, ures)
